# baseline (speedup 1.0000x reference)
import functools

import jax
import jax.numpy as jnp
from jax import lax
from jax.experimental import pallas as pl
from jax.experimental.pallas import tpu as pltpu
from jax.experimental.pallas import tpu_sc as plsc

_NC = 2
_NS = 16
_NW = _NC * _NS
_DP = 128


def _gather_t(x_t, table, N, J, D):
    nblk = N // _DP // _NW
    mesh = plsc.VectorSubcoreMesh(
        core_axis_name="c", subcore_axis_name="s",
        num_cores=_NC, num_subcores=_NS)

    @functools.partial(
        pl.kernel,
        out_type=jax.ShapeDtypeStruct((J, D, N), jnp.float32),
        mesh=mesh,
        scratch_types=[
            pltpu.VMEM((J, _DP), jnp.int32),
            pltpu.VMEM((2, _DP, _DP), jnp.float32),
            pltpu.VMEM((2, D, _DP), jnp.float32),
            [pltpu.SemaphoreType.DMA] * 2,
            [pltpu.SemaphoreType.DMA] * 2,
        ],
        compiler_params=pltpu.CompilerParams(
            use_tc_tiling_on_sc=True, needs_layout_passes=False),
    )
    def k(x_hbm, w_hbm, out_hbm, idx_v, rows_v, tiles_v, semg, sems):
        wid = lax.axis_index("s") * _NC + lax.axis_index("c")
        iota = lax.iota(jnp.int32, 16)
        rowv = [iota + (icg * 16) for icg in range(8)]

        def fire_g(u, b):
            pltpu.async_copy(w_hbm.at[idx_v.at[u]], rows_v.at[b], semg[b])

        def wait_g(b):
            pltpu.make_async_copy(
                w_hbm.at[idx_v.at[0]], rows_v.at[b], semg[b]).wait()

        def fire_s(u, b, i0):
            pltpu.async_copy(
                tiles_v.at[b], out_hbm.at[u, :, pl.ds(i0, _DP)], sems[b])

        def wait_s(b, i0):
            pltpu.make_async_copy(
                tiles_v.at[b], out_hbm.at[0, :, pl.ds(i0, _DP)],
                sems[b]).wait()

        def transpose(b):
            @plsc.parallel_loop(0, D, unroll=8)
            def _col(kk):
                colv = jnp.full((16,), 0, jnp.int32) + kk
                for icg in range(8):
                    v = plsc.load_gather(rows_v.at[b], [rowv[icg], colv])
                    tiles_v[b, kk, pl.ds(icg * 16, 16)] = v

        for blk in range(nblk):
            i0 = (wid * nblk + blk) * _DP
            pltpu.sync_copy(x_hbm.at[:, pl.ds(i0, _DP)], idx_v)
            fire_g(0, 0)
            fire_g(1, 1)
            for u in range(2):
                wait_g(u % 2)
                transpose(u % 2)
                fire_s(u, u % 2, i0)
                fire_g(u + 2, u % 2)

            @pl.loop(2, J - 2)
            def _unit(u):
                for b in range(2):
                    @pl.when((u % 2) == b)
                    def _():
                        wait_g(b)
                        wait_s(b, i0)
                        transpose(b)
                        fire_s(u, b, i0)
                        fire_g(u + 2, b)

            for uu in range(J - 2, J):
                b = uu % 2
                wait_g(b)
                wait_s(b, i0)
                transpose(b)
                fire_s(uu, b, i0)
            for b in range(2):
                wait_s(b, i0)

    return k(x_t, table)


def kernel(x, W):
    N, J = x.shape
    D = W.shape[1]
    x_t = x.T.astype(jnp.int32)
    W_pad = jnp.pad(W, ((0, 0), (0, _DP - D)))
    P = _gather_t(x_t, W_pad, N, J, D)
    return P.transpose(2, 0, 1)


# speedup vs baseline: 1.0022x; 1.0022x over previous
import functools

import jax
import jax.numpy as jnp
from jax import lax
from jax.experimental import pallas as pl
from jax.experimental.pallas import tpu as pltpu
from jax.experimental.pallas import tpu_sc as plsc

_NC = 2
_NS = 16
_NW = _NC * _NS
_DP = 128


def _gather_t(x_t, table, N, J, D):
    nblk = N // _DP // _NW
    mesh = plsc.VectorSubcoreMesh(
        core_axis_name="c", subcore_axis_name="s",
        num_cores=_NC, num_subcores=_NS)

    @functools.partial(
        pl.kernel,
        out_type=jax.ShapeDtypeStruct((J, D, N), jnp.float32),
        mesh=mesh,
        scratch_types=[
            pltpu.VMEM((J, _DP), jnp.int32),
            pltpu.VMEM((2, _DP, _DP), jnp.float32),
            pltpu.VMEM((2, D, _DP), jnp.float32),
            [pltpu.SemaphoreType.DMA] * 2,
            [pltpu.SemaphoreType.DMA] * 2,
        ],
        compiler_params=pltpu.CompilerParams(
            use_tc_tiling_on_sc=True, needs_layout_passes=False,
            disable_bounds_checks=True),
    )
    def k(x_hbm, w_hbm, out_hbm, idx_v, rows_v, tiles_v, semg, sems):
        wid = lax.axis_index("s") * _NC + lax.axis_index("c")
        iota = lax.iota(jnp.int32, 16)
        rowv = [iota + (icg * 16) for icg in range(8)]

        def fire_g(u, b):
            pltpu.async_copy(w_hbm.at[idx_v.at[u]], rows_v.at[b], semg[b])

        def wait_g(b):
            pltpu.make_async_copy(
                w_hbm.at[idx_v.at[0]], rows_v.at[b], semg[b]).wait()

        def fire_s(u, b, i0):
            pltpu.async_copy(
                tiles_v.at[b], out_hbm.at[u, :, pl.ds(i0, _DP)], sems[b])

        def wait_s(b, i0):
            pltpu.make_async_copy(
                tiles_v.at[b], out_hbm.at[0, :, pl.ds(i0, _DP)],
                sems[b]).wait()

        def transpose(b):
            @plsc.parallel_loop(0, D, unroll=8)
            def _col(kk):
                colv = jnp.full((16,), 0, jnp.int32) + kk
                for icg in range(8):
                    v = plsc.load_gather(rows_v.at[b], [rowv[icg], colv])
                    tiles_v[b, kk, pl.ds(icg * 16, 16)] = v

        for blk in range(nblk):
            i0 = (wid * nblk + blk) * _DP
            pltpu.sync_copy(x_hbm.at[:, pl.ds(i0, _DP)], idx_v)
            fire_g(0, 0)
            fire_g(1, 1)
            for u in range(2):
                wait_g(u % 2)
                transpose(u % 2)
                fire_s(u, u % 2, i0)
                fire_g(u + 2, u % 2)

            @pl.loop(2, J - 2)
            def _unit(u):
                for b in range(2):
                    @pl.when((u % 2) == b)
                    def _():
                        wait_g(b)
                        wait_s(b, i0)
                        transpose(b)
                        fire_s(u, b, i0)
                        fire_g(u + 2, b)

            for uu in range(J - 2, J):
                b = uu % 2
                wait_g(b)
                wait_s(b, i0)
                transpose(b)
                fire_s(uu, b, i0)
            for b in range(2):
                wait_s(b, i0)

    return k(x_t, table)


def kernel(x, W):
    N, J = x.shape
    D = W.shape[1]
    x_t = x.T.astype(jnp.int32)
    W_pad = jnp.pad(W, ((0, 0), (0, _DP - D)))
    P = _gather_t(x_t, W_pad, N, J, D)
    return P.transpose(2, 0, 1)


# speedup vs baseline: 1.3719x; 1.3689x over previous
import functools

import jax
import jax.numpy as jnp
from jax import lax
from jax.experimental import pallas as pl
from jax.experimental.pallas import tpu as pltpu
from jax.experimental.pallas import tpu_sc as plsc

_NC = 2
_NS = 16
_NW = _NC * _NS

_SUB = 128
_NSUB = 2
_CHUNK = _SUB * _NSUB
_NBUF = 3
_DP = 128


def _gather(idx2d, table, B):
    b_per_w = B // _NW
    n_chunks = b_per_w // _CHUNK
    idxrows_per_w = b_per_w // _SUB

    mesh = plsc.VectorSubcoreMesh(
        core_axis_name="c", subcore_axis_name="s",
        num_cores=_NC, num_subcores=_NS)

    @functools.partial(
        pl.kernel,
        out_type=jax.ShapeDtypeStruct((B, _DP), jnp.float32),
        mesh=mesh,
        scratch_types=[
            pltpu.VMEM((_NBUF, _NSUB, _SUB), jnp.int32),
            pltpu.VMEM((_NBUF, _CHUNK, _DP), jnp.float32),
            [pltpu.SemaphoreType.DMA] * _NBUF,
            [pltpu.SemaphoreType.DMA] * _NBUF,
            [pltpu.SemaphoreType.DMA] * _NBUF,
        ],
        compiler_params=pltpu.CompilerParams(use_tc_tiling_on_sc=False),
    )
    def k(idx_hbm, table_hbm, out_hbm, idx_v, rows_v, semi, semg, semo):
        wid = lax.axis_index("s") * _NC + lax.axis_index("c")
        idxrow0 = wid * idxrows_per_w
        row0 = wid * b_per_w

        def start_idx(b, g):
            pltpu.async_copy(
                idx_hbm.at[pl.ds(idxrow0 + g * _NSUB, _NSUB)],
                idx_v.at[b], semi[b])

        def wait_idx(b):
            pltpu.make_async_copy(
                idx_hbm.at[pl.ds(idxrow0, _NSUB)],
                idx_v.at[b], semi[b]).wait()

        def start_gather(b):
            for j in range(_NSUB):
                pltpu.async_copy(
                    table_hbm.at[idx_v.at[b, j]],
                    rows_v.at[b, pl.ds(j * _SUB, _SUB)],
                    semg[b])

        def wait_gather(b):
            for j in range(_NSUB):
                pltpu.make_async_copy(
                    table_hbm.at[idx_v.at[b, j]],
                    rows_v.at[b, pl.ds(j * _SUB, _SUB)],
                    semg[b]).wait()

        def start_out(b, g):
            pltpu.async_copy(
                rows_v.at[b],
                out_hbm.at[pl.ds(row0 + g * _CHUNK, _CHUNK)],
                semo[b])

        def wait_out(b):
            pltpu.make_async_copy(
                rows_v.at[b],
                out_hbm.at[pl.ds(row0, _CHUNK)],
                semo[b]).wait()


        for b in range(_NBUF):
            start_idx(b, b)
        wait_idx(0)
        start_gather(0)

        for g in range(1, _NBUF):
            b, bp = g % _NBUF, (g - 1) % _NBUF
            wait_idx(b)
            start_gather(b)
            wait_gather(bp)
            start_idx(bp, g - 1 + _NBUF)
            start_out(bp, g - 1)

        @pl.loop(_NBUF, n_chunks - 2)
        def _step(g):
            for b in range(_NBUF):
                bp = (b - 1) % _NBUF

                @pl.when((g % _NBUF) == b)
                def _():
                    wait_idx(b)
                    wait_out(b)
                    start_gather(b)
                    wait_gather(bp)
                    start_idx(bp, g - 1 + _NBUF)
                    start_out(bp, g - 1)

        for g in range(n_chunks - 2, n_chunks):
            b, bp = g % _NBUF, (g - 1) % _NBUF
            wait_idx(b)
            wait_out(b)
            start_gather(b)
            wait_gather(bp)
            start_out(bp, g - 1)
        bl = (n_chunks - 1) % _NBUF
        wait_gather(bl)
        start_out(bl, n_chunks - 1)
        for b in range(_NBUF):
            wait_out(b)

    return k(idx2d, table)


def kernel(x, W):
    N, J = x.shape
    D = W.shape[1]
    B = N * J
    idx2d = x.reshape(B // _SUB, _SUB).astype(jnp.int32)
    W_pad = jnp.pad(W, ((0, 0), (0, _DP - D)))
    out2 = _gather(idx2d, W_pad, B)
    return out2[:, :D].reshape(N, J, D)
